# baseline (speedup 1.0000x reference)
import jax
import jax.numpy as jnp
from jax.experimental import pallas as pl
from jax.experimental.pallas import tpu as pltpu

B = 1
S = 2048
D = 768
N = 4096
K = 32
TBLK = 256
GRID = S // TBLK


def _f2k(v):
    b = jax.lax.bitcast_convert_type(v, jnp.int32)
    return jnp.where(b < 0, b ^ 0x7FFFFFFF, b)


def _k2f(key):
    b = jnp.where(key < 0, key ^ 0x7FFFFFFF, key)
    return jax.lax.bitcast_convert_type(b, jnp.float32)


def _score_threshold(scores, k):
    kf = float(k)
    q = 3.0
    gm = scores[:, 0:128]
    for c in range(1, N // 128):
        gm = jnp.maximum(gm, scores[:, c * 128:(c + 1) * 128])
    gm2 = jnp.maximum(jnp.maximum(gm[:, 0:32], gm[:, 32:64]),
                      jnp.maximum(gm[:, 64:96], gm[:, 96:128]))
    lo_f = jnp.min(gm2, axis=1, keepdims=True)
    lo_k = _f2k(lo_f)
    hi_k = _f2k(jnp.max(gm2, axis=1, keepdims=True)) + 1
    c_lo = jnp.full_like(lo_f, float(N))

    def cond(carry):
        lo_k, hi_k, c_lo = carry
        return jnp.any(~((hi_k - lo_k <= 1) | (c_lo <= kf + q)))

    def probe(carry):
        lo_k, hi_k, c_lo = carry
        mid_k = (lo_k & hi_k) + ((lo_k ^ hi_k) >> 1)
        t_f = _k2f(mid_k)
        c = jnp.sum((scores >= t_f).astype(jnp.float32), axis=1, keepdims=True)
        ok = c >= kf
        return (jnp.where(ok, mid_k, lo_k), jnp.where(ok, hi_k, mid_k),
                jnp.where(ok, c, c_lo))

    def body(carry):
        return probe(probe(probe(carry)))

    lo_k, hi_k, c_lo = jax.lax.while_loop(cond, body, (lo_k, hi_k, c_lo))
    thr_a = _k2f(lo_k)
    inf = jnp.float32(jnp.inf)
    m0 = jnp.min(jnp.where(scores >= thr_a, scores, inf), axis=1, keepdims=True)
    m1 = jnp.min(jnp.where(scores > m0, scores, inf), axis=1, keepdims=True)
    m2 = jnp.min(jnp.where(scores > m1, scores, inf), axis=1, keepdims=True)
    m3 = jnp.min(jnp.where(scores > m2, scores, inf), axis=1, keepdims=True)
    r = c_lo - kf
    chain = jnp.where(r < 0.5, m0,
                      jnp.where(r < 1.5, m1, jnp.where(r < 2.5, m2, m3)))
    return jnp.where(hi_k - lo_k <= 1, thr_a, chain)


def _apply_gate(scores, tau, thr):
    raw = scores - tau
    thr2 = jnp.maximum(thr, _k2f(_f2k(tau) + 1))
    keep = scores >= thr2
    eg = jnp.where(keep, jnp.exp(raw) - 1.0, 0.0)
    s = jnp.sum(eg, axis=1, keepdims=True) + 1e-08
    m = jnp.max(eg, axis=1, keepdims=True)
    return eg / s * jnp.tanh(m)


def _aux_of(sum_ref):
    mean = sum_ref[...] / float(B * S)
    return jnp.sum((mean - 1.0 / N) ** 2) * float(N)


def _qk_kernel(x_ref, emb_ref, w_ref, b_ref, gq_ref, gk_ref, aux_ref,
               sc_ref, tau_ref, sq_ref, sk_ref):
    i = pl.program_id(0)

    @pl.when(i < GRID)
    def _():
        xb = x_ref[...]
        sc_ref[i % 2] = jax.lax.dot_general(
            xb / 0.9, emb_ref[...], dimension_numbers=(((1,), (1,)), ((), ())),
            preferred_element_type=jnp.float32)
        tau_ref[i % 2] = jnp.dot(
            xb, w_ref[...], preferred_element_type=jnp.float32) + b_ref[...]

    @pl.when(i == 1)
    def _():
        sq_ref[...] = jnp.zeros_like(sq_ref)
        sk_ref[...] = jnp.zeros_like(sk_ref)

    @pl.when(i > 0)
    def _():
        j = (i - 1) % 2
        scores = sc_ref[j]
        tau = tau_ref[j]
        thr = _score_threshold(scores, K)
        gq = _apply_gate(scores, tau[:, 0:1], thr)
        gk = _apply_gate(scores, tau[:, 1:2], thr)
        gq_ref[...] = gq
        gk_ref[...] = gk
        sq_ref[...] += jnp.sum(gq, axis=0, keepdims=True)
        sk_ref[...] += jnp.sum(gk, axis=0, keepdims=True)

    @pl.when(i == GRID)
    def _():
        aux_ref[...] = jnp.reshape(_aux_of(sq_ref) + _aux_of(sk_ref), (1, 1))


def _single_kernel(x_ref, emb_ref, w_ref, b_ref, auxin_ref, g_ref, aux_ref,
                   sc_ref, tau_ref, s_ref):
    i = pl.program_id(0)

    @pl.when(i < GRID)
    def _():
        xb = x_ref[...]
        sc_ref[i % 2] = jax.lax.dot_general(
            xb / 0.9, emb_ref[...], dimension_numbers=(((1,), (1,)), ((), ())),
            preferred_element_type=jnp.float32)
        tau_ref[i % 2] = jnp.dot(
            xb, w_ref[...], preferred_element_type=jnp.float32) + b_ref[...]

    @pl.when(i == 1)
    def _():
        s_ref[...] = jnp.zeros_like(s_ref)

    @pl.when(i > 0)
    def _():
        j = (i - 1) % 2
        scores = sc_ref[j]
        g = _apply_gate(scores, tau_ref[j], _score_threshold(scores, K))
        g_ref[...] = g
        s_ref[...] += jnp.sum(g, axis=0, keepdims=True)

    @pl.when(i == GRID)
    def _():
        aux_ref[...] = auxin_ref[...] + jnp.reshape(_aux_of(s_ref), (1, 1))


def kernel(x, qk_emb, v_emb, know_emb, W_tau_attn, b_tau_attn, W_tau_know,
           b_tau_know):
    x2 = x.reshape(S, D)
    last = GRID - 1
    x_spec = pl.BlockSpec((TBLK, D), lambda i: (jnp.minimum(i, last), 0))
    emb_spec = pl.BlockSpec((N, D), lambda i: (0, 0))
    g_spec = pl.BlockSpec((TBLK, N), lambda i: (jnp.where(i > 0, i - 1, 0), 0))
    aux_spec = pl.BlockSpec((1, 1), lambda i: (0, 0))
    b3 = b_tau_attn.reshape(1, 3)
    sum_scratch = pltpu.VMEM((1, N), jnp.float32)
    sc_scratch = pltpu.VMEM((2, TBLK, N), jnp.float32)

    gq, gk, aux_qk = pl.pallas_call(
        _qk_kernel,
        grid=(GRID + 1,),
        in_specs=[x_spec, emb_spec, pl.BlockSpec((D, 3), lambda i: (0, 0)),
                  pl.BlockSpec((1, 3), lambda i: (0, 0))],
        out_specs=[g_spec, g_spec, aux_spec],
        out_shape=[
            jax.ShapeDtypeStruct((S, N), jnp.float32),
            jax.ShapeDtypeStruct((S, N), jnp.float32),
            jax.ShapeDtypeStruct((1, 1), jnp.float32),
        ],
        scratch_shapes=[sc_scratch, pltpu.VMEM((2, TBLK, 3), jnp.float32),
                        sum_scratch, sum_scratch],
    )(x2, qk_emb, W_tau_attn, b3)

    def _single(emb, w1, b1, auxin):
        return pl.pallas_call(
            _single_kernel,
            grid=(GRID + 1,),
            in_specs=[x_spec, emb_spec, pl.BlockSpec((D, 1), lambda i: (0, 0)),
                      pl.BlockSpec((1, 1), lambda i: (0, 0)), aux_spec],
            out_specs=[g_spec, aux_spec],
            out_shape=[
                jax.ShapeDtypeStruct((S, N), jnp.float32),
                jax.ShapeDtypeStruct((1, 1), jnp.float32),
            ],
            scratch_shapes=[sc_scratch, pltpu.VMEM((2, TBLK, 1), jnp.float32),
                            sum_scratch],
        )(x2, emb, w1, b1, auxin)

    gv, aux_attn = _single(v_emb, W_tau_attn[:, 2:3],
                           b_tau_attn[2].reshape(1, 1), aux_qk)
    gknow, aux_know = _single(know_emb, W_tau_know, b_tau_know.reshape(1, 1),
                              jnp.zeros((1, 1), jnp.float32))

    shape = (B, S, N)
    return (gq.reshape(shape), gk.reshape(shape), gv.reshape(shape),
            aux_attn.reshape(()), gknow.reshape(shape), aux_know.reshape(()))


# speedup vs baseline: 1.0101x; 1.0101x over previous
import jax
import jax.numpy as jnp
from jax.experimental import pallas as pl
from jax.experimental.pallas import tpu as pltpu

B = 1
S = 2048
D = 768
N = 4096
K = 32
TBLK = 256
GRID = S // TBLK


def _f2k(v):
    b = jax.lax.bitcast_convert_type(v, jnp.int32)
    return jnp.where(b < 0, b ^ 0x7FFFFFFF, b)


def _k2f(key):
    b = jnp.where(key < 0, key ^ 0x7FFFFFFF, key)
    return jax.lax.bitcast_convert_type(b, jnp.float32)


def _score_threshold(scores, k):
    kf = float(k)
    q = 3.0
    gm = scores[:, 0:128]
    for c in range(1, N // 128):
        gm = jnp.maximum(gm, scores[:, c * 128:(c + 1) * 128])
    gm2 = jnp.maximum(jnp.maximum(gm[:, 0:32], gm[:, 32:64]),
                      jnp.maximum(gm[:, 64:96], gm[:, 96:128]))
    lo_f = jnp.min(gm2, axis=1, keepdims=True)
    lo_k = _f2k(lo_f)
    hi_k = _f2k(jnp.max(gm2, axis=1, keepdims=True)) + 1
    c_lo = jnp.full_like(lo_f, float(N))

    def cond(carry):
        lo_k, hi_k, c_lo = carry
        return jnp.any(~((hi_k - lo_k <= 1) | (c_lo <= kf + q)))

    def probe(carry):
        lo_k, hi_k, c_lo = carry
        mid_k = (lo_k & hi_k) + ((lo_k ^ hi_k) >> 1)
        t_f = _k2f(mid_k)
        c = jnp.sum((scores >= t_f).astype(jnp.float32), axis=1, keepdims=True)
        ok = c >= kf
        return (jnp.where(ok, mid_k, lo_k), jnp.where(ok, hi_k, mid_k),
                jnp.where(ok, c, c_lo))

    def body(carry):
        return probe(probe(carry))

    lo_k, hi_k, c_lo = jax.lax.while_loop(cond, body, (lo_k, hi_k, c_lo))
    thr_a = _k2f(lo_k)
    inf = jnp.float32(jnp.inf)
    m0 = jnp.min(jnp.where(scores >= thr_a, scores, inf), axis=1, keepdims=True)
    m1 = jnp.min(jnp.where(scores > m0, scores, inf), axis=1, keepdims=True)
    m2 = jnp.min(jnp.where(scores > m1, scores, inf), axis=1, keepdims=True)
    m3 = jnp.min(jnp.where(scores > m2, scores, inf), axis=1, keepdims=True)
    r = c_lo - kf
    chain = jnp.where(r < 0.5, m0,
                      jnp.where(r < 1.5, m1, jnp.where(r < 2.5, m2, m3)))
    return jnp.where(hi_k - lo_k <= 1, thr_a, chain)


def _apply_gate(scores, tau, thr):
    raw = scores - tau
    thr2 = jnp.maximum(thr, _k2f(_f2k(tau) + 1))
    keep = scores >= thr2
    eg = jnp.where(keep, jnp.exp(raw) - 1.0, 0.0)
    s = jnp.sum(eg, axis=1, keepdims=True) + 1e-08
    m = jnp.max(eg, axis=1, keepdims=True)
    return eg / s * jnp.tanh(m)


def _aux_of(sum_ref):
    mean = sum_ref[...] / float(B * S)
    return jnp.sum((mean - 1.0 / N) ** 2) * float(N)


def _qk_kernel(x_ref, emb_ref, w_ref, b_ref, gq_ref, gk_ref, aux_ref,
               sc_ref, tau_ref, sq_ref, sk_ref):
    i = pl.program_id(0)

    @pl.when(i < GRID)
    def _():
        xb = x_ref[...]
        sc_ref[i % 2] = jax.lax.dot_general(
            xb / 0.9, emb_ref[...], dimension_numbers=(((1,), (1,)), ((), ())),
            preferred_element_type=jnp.float32)
        tau_ref[i % 2] = jnp.dot(
            xb, w_ref[...], preferred_element_type=jnp.float32) + b_ref[...]

    @pl.when(i == 1)
    def _():
        sq_ref[...] = jnp.zeros_like(sq_ref)
        sk_ref[...] = jnp.zeros_like(sk_ref)

    @pl.when(i > 0)
    def _():
        j = (i - 1) % 2
        scores = sc_ref[j]
        tau = tau_ref[j]
        thr = _score_threshold(scores, K)
        gq = _apply_gate(scores, tau[:, 0:1], thr)
        gk = _apply_gate(scores, tau[:, 1:2], thr)
        gq_ref[...] = gq
        gk_ref[...] = gk
        sq_ref[...] += jnp.sum(gq, axis=0, keepdims=True)
        sk_ref[...] += jnp.sum(gk, axis=0, keepdims=True)

    @pl.when(i == GRID)
    def _():
        aux_ref[...] = jnp.reshape(_aux_of(sq_ref) + _aux_of(sk_ref), (1, 1))


def _single_kernel(x_ref, emb_ref, w_ref, b_ref, auxin_ref, g_ref, aux_ref,
                   sc_ref, tau_ref, s_ref):
    i = pl.program_id(0)

    @pl.when(i < GRID)
    def _():
        xb = x_ref[...]
        sc_ref[i % 2] = jax.lax.dot_general(
            xb / 0.9, emb_ref[...], dimension_numbers=(((1,), (1,)), ((), ())),
            preferred_element_type=jnp.float32)
        tau_ref[i % 2] = jnp.dot(
            xb, w_ref[...], preferred_element_type=jnp.float32) + b_ref[...]

    @pl.when(i == 1)
    def _():
        s_ref[...] = jnp.zeros_like(s_ref)

    @pl.when(i > 0)
    def _():
        j = (i - 1) % 2
        scores = sc_ref[j]
        g = _apply_gate(scores, tau_ref[j], _score_threshold(scores, K))
        g_ref[...] = g
        s_ref[...] += jnp.sum(g, axis=0, keepdims=True)

    @pl.when(i == GRID)
    def _():
        aux_ref[...] = auxin_ref[...] + jnp.reshape(_aux_of(s_ref), (1, 1))


def kernel(x, qk_emb, v_emb, know_emb, W_tau_attn, b_tau_attn, W_tau_know,
           b_tau_know):
    x2 = x.reshape(S, D)
    last = GRID - 1
    x_spec = pl.BlockSpec((TBLK, D), lambda i: (jnp.minimum(i, last), 0))
    emb_spec = pl.BlockSpec((N, D), lambda i: (0, 0))
    g_spec = pl.BlockSpec((TBLK, N), lambda i: (jnp.where(i > 0, i - 1, 0), 0))
    aux_spec = pl.BlockSpec((1, 1), lambda i: (0, 0))
    b3 = b_tau_attn.reshape(1, 3)
    sum_scratch = pltpu.VMEM((1, N), jnp.float32)
    sc_scratch = pltpu.VMEM((2, TBLK, N), jnp.float32)

    gq, gk, aux_qk = pl.pallas_call(
        _qk_kernel,
        grid=(GRID + 1,),
        in_specs=[x_spec, emb_spec, pl.BlockSpec((D, 3), lambda i: (0, 0)),
                  pl.BlockSpec((1, 3), lambda i: (0, 0))],
        out_specs=[g_spec, g_spec, aux_spec],
        out_shape=[
            jax.ShapeDtypeStruct((S, N), jnp.float32),
            jax.ShapeDtypeStruct((S, N), jnp.float32),
            jax.ShapeDtypeStruct((1, 1), jnp.float32),
        ],
        scratch_shapes=[sc_scratch, pltpu.VMEM((2, TBLK, 3), jnp.float32),
                        sum_scratch, sum_scratch],
    )(x2, qk_emb, W_tau_attn, b3)

    def _single(emb, w1, b1, auxin):
        return pl.pallas_call(
            _single_kernel,
            grid=(GRID + 1,),
            in_specs=[x_spec, emb_spec, pl.BlockSpec((D, 1), lambda i: (0, 0)),
                      pl.BlockSpec((1, 1), lambda i: (0, 0)), aux_spec],
            out_specs=[g_spec, aux_spec],
            out_shape=[
                jax.ShapeDtypeStruct((S, N), jnp.float32),
                jax.ShapeDtypeStruct((1, 1), jnp.float32),
            ],
            scratch_shapes=[sc_scratch, pltpu.VMEM((2, TBLK, 1), jnp.float32),
                            sum_scratch],
        )(x2, emb, w1, b1, auxin)

    gv, aux_attn = _single(v_emb, W_tau_attn[:, 2:3],
                           b_tau_attn[2].reshape(1, 1), aux_qk)
    gknow, aux_know = _single(know_emb, W_tau_know, b_tau_know.reshape(1, 1),
                              jnp.zeros((1, 1), jnp.float32))

    shape = (B, S, N)
    return (gq.reshape(shape), gk.reshape(shape), gv.reshape(shape),
            aux_attn.reshape(()), gknow.reshape(shape), aux_know.reshape(()))
